# baseline (speedup 1.0000x reference)
import jax
import jax.numpy as jnp
from jax.experimental import pallas as pl
from jax.experimental.pallas import tpu as pltpu

_TILE_N = 2048


def _head_kernel(x_ref, wc_ref, bc_ref, wr_ref, br_ref, wd_ref, bd_ref,
                 oc_ref, og_ref, od_ref):
    x = x_ref[0]
    oc_ref[0] = jnp.dot(wc_ref[:], x, preferred_element_type=jnp.float32) + bc_ref[:]
    og_ref[0] = jnp.dot(wr_ref[:], x, preferred_element_type=jnp.float32) + br_ref[:]
    od_ref[0] = jnp.dot(wd_ref[:], x, preferred_element_type=jnp.float32) + bd_ref[:]


@jax.jit
def kernel(x, W_cls, b_cls, W_reg, b_reg, W_dir, b_dir):
    B, C, H, W = x.shape
    HW = H * W
    Oc = W_cls.shape[0]
    Og = W_reg.shape[0]
    Od = W_dir.shape[0]
    xf = x.reshape(B, C, HW)
    n_tiles = pl.cdiv(HW, _TILE_N)

    def w_spec(o):
        return pl.BlockSpec((o, C), lambda b, n: (0, 0))

    def b_spec(o):
        return pl.BlockSpec((o, 1), lambda b, n: (0, 0))

    def o_spec(o):
        return pl.BlockSpec((1, o, _TILE_N), lambda b, n: (b, 0, n))

    out_cls, out_reg, out_dir = pl.pallas_call(
        _head_kernel,
        grid=(B, n_tiles),
        in_specs=[
            pl.BlockSpec((1, C, _TILE_N), lambda b, n: (b, 0, n)),
            w_spec(Oc), b_spec(Oc),
            w_spec(Og), b_spec(Og),
            w_spec(Od), b_spec(Od),
        ],
        out_specs=(o_spec(Oc), o_spec(Og), o_spec(Od)),
        out_shape=(
            jax.ShapeDtypeStruct((B, Oc, HW), jnp.float32),
            jax.ShapeDtypeStruct((B, Og, HW), jnp.float32),
            jax.ShapeDtypeStruct((B, Od, HW), jnp.float32),
        ),
        compiler_params=pltpu.CompilerParams(
            dimension_semantics=("parallel", "parallel"),
        ),
    )(xf, W_cls, b_cls.reshape(Oc, 1), W_reg, b_reg.reshape(Og, 1),
      W_dir, b_dir.reshape(Od, 1))

    return (out_cls.reshape(B, Oc, H, W),
            out_reg.reshape(B, Og, H, W),
            out_dir.reshape(B, Od, H, W))


# speedup vs baseline: 1.0792x; 1.0792x over previous
import jax
import jax.numpy as jnp
from jax.experimental import pallas as pl
from jax.experimental.pallas import tpu as pltpu

_TILE_N = 8192


def _head_kernel(x_ref, wc_ref, bc_ref, wr_ref, br_ref, wd_ref, bd_ref,
                 oc_ref, og_ref, od_ref):
    x = x_ref[0]
    oc_ref[0] = jnp.dot(wc_ref[:], x, preferred_element_type=jnp.float32) + bc_ref[:]
    og_ref[0] = jnp.dot(wr_ref[:], x, preferred_element_type=jnp.float32) + br_ref[:]
    od_ref[0] = jnp.dot(wd_ref[:], x, preferred_element_type=jnp.float32) + bd_ref[:]


@jax.jit
def kernel(x, W_cls, b_cls, W_reg, b_reg, W_dir, b_dir):
    B, C, H, W = x.shape
    HW = H * W
    Oc = W_cls.shape[0]
    Og = W_reg.shape[0]
    Od = W_dir.shape[0]
    xf = x.reshape(B, C, HW)
    n_tiles = pl.cdiv(HW, _TILE_N)

    def w_spec(o):
        return pl.BlockSpec((o, C), lambda b, n: (0, 0))

    def b_spec(o):
        return pl.BlockSpec((o, 1), lambda b, n: (0, 0))

    def o_spec(o):
        return pl.BlockSpec((1, o, _TILE_N), lambda b, n: (b, 0, n))

    out_cls, out_reg, out_dir = pl.pallas_call(
        _head_kernel,
        grid=(B, n_tiles),
        in_specs=[
            pl.BlockSpec((1, C, _TILE_N), lambda b, n: (b, 0, n)),
            w_spec(Oc), b_spec(Oc),
            w_spec(Og), b_spec(Og),
            w_spec(Od), b_spec(Od),
        ],
        out_specs=(o_spec(Oc), o_spec(Og), o_spec(Od)),
        out_shape=(
            jax.ShapeDtypeStruct((B, Oc, HW), jnp.float32),
            jax.ShapeDtypeStruct((B, Og, HW), jnp.float32),
            jax.ShapeDtypeStruct((B, Od, HW), jnp.float32),
        ),
        compiler_params=pltpu.CompilerParams(
            dimension_semantics=("parallel", "parallel"),
        ),
    )(xf, W_cls, b_cls.reshape(Oc, 1), W_reg, b_reg.reshape(Og, 1),
      W_dir, b_dir.reshape(Od, 1))

    return (out_cls.reshape(B, Oc, H, W),
            out_reg.reshape(B, Og, H, W),
            out_dir.reshape(B, Od, H, W))
